# Initial kernel scaffold; baseline (speedup 1.0000x reference)
import jax
import jax.numpy as jnp
from jax.experimental import pallas as pl


def kernel(geom, geom_template_posed, nbs_idxs, nbs_weights):
    raise NotImplementedError("write your pallas kernel here")


# speedup vs baseline: 18.8436x; 18.8436x over previous
import functools

import jax
import jax.numpy as jnp
from jax import lax
from jax.experimental import pallas as pl
from jax.experimental.pallas import tpu as pltpu
from jax.experimental.pallas import tpu_sc as plsc

B, N, C, K = 4, 100000, 3, 16
L = 16
NC, NS = 2, 16
NW = NC * NS
NPAD = 102400

CH = 128
NCHUNKS = NPAD // CH
TRIPS = NCHUNKS // NW

_params = pltpu.CompilerParams(needs_layout_passes=False,
                               use_tc_tiling_on_sc=False)
_mesh = plsc.VectorSubcoreMesh(core_axis_name="c", subcore_axis_name="s")


def _wid():
    return lax.axis_index("s") * NC + lax.axis_index("c")


@functools.partial(
    pl.kernel,
    mesh=_mesh,
    compiler_params=_params,
    out_type=jax.ShapeDtypeStruct((NPAD, L), jnp.float32),
    scratch_types=[
        pltpu.VMEM((B, CH * C), jnp.float32),
        pltpu.VMEM((B, CH * C), jnp.float32),
        pltpu.VMEM((CH, L), jnp.float32),
    ],
)
def _build_table(geom_hbm, gtp_hbm, table_hbm, g_v, t_v, tab_v):
    w = _wid()

    def trip_body(t, carry):
        cid = w + t * NW
        cbase = cid * CH
        pltpu.sync_copy(geom_hbm.at[:, pl.ds(cbase * C, CH * C)], g_v)
        pltpu.sync_copy(gtp_hbm.at[:, pl.ds(cbase * C, CH * C)], t_v)

        def node(i, c2):
            lane = lax.iota(jnp.int32, L)
            valid = lane < (B * C)
            b_idx = jnp.where(valid, lane // C, 0)
            c_idx = jnp.where(valid, lane % C, 0)
            e = jnp.full((L,), i * C, jnp.int32) + c_idx
            g = plsc.load_gather(g_v, [b_idx, e])
            tt = plsc.load_gather(t_v, [b_idx, e])
            tab_v[i] = jnp.where(valid, g - tt, jnp.zeros((L,), jnp.float32))
            return c2

        lax.fori_loop(0, CH, node, 0)
        pltpu.sync_copy(tab_v, table_hbm.at[pl.ds(cbase, CH)])
        return carry

    lax.fori_loop(0, TRIPS, trip_body, 0)


@functools.partial(
    pl.kernel,
    mesh=_mesh,
    compiler_params=_params,
    out_type=jax.ShapeDtypeStruct((NW * L,), jnp.float32),
    scratch_types=[
        pltpu.VMEM((K, CH), jnp.int32),
        pltpu.VMEM((CH * K, L), jnp.float32),
        pltpu.VMEM((CH, L), jnp.float32),
        pltpu.VMEM((CH, L), jnp.float32),
        pltpu.VMEM((L,), jnp.float32),
        pltpu.SemaphoreType.DMA,
    ],
)
def _loss_partials(table_hbm, idx3_hbm, w_hbm, out_hbm,
                   idx_v, rows_v, w_v, own_v, part_v, sem):
    w = _wid()

    def chunk(c, sq):
        crow = w * TRIPS + c
        cbase = crow * CH
        pltpu.sync_copy(idx3_hbm.at[crow], idx_v)
        handles = [
            pltpu.async_copy(table_hbm.at[idx_v.at[g]],
                             rows_v.at[pl.ds(g * CH, CH)], sem)
            for g in range(K)
        ]
        pltpu.sync_copy(w_hbm.at[pl.ds(cbase, CH)], w_v)
        pltpu.sync_copy(table_hbm.at[pl.ds(cbase, CH)], own_v)
        for h in handles:
            h.wait()

        def node(i, s):
            acc = own_v[i]
            wv = w_v[i]
            base = i * K
            for k in range(K):
                acc = acc + wv[k] * rows_v[base + k]
            return s + acc * acc

        return lax.fori_loop(0, CH, node, sq)

    sq = lax.fori_loop(0, TRIPS, chunk, jnp.zeros((L,), jnp.float32))
    part_v[...] = sq
    pltpu.sync_copy(part_v, out_hbm.at[pl.ds(w * L, L)])


def kernel(geom, geom_template_posed, nbs_idxs, nbs_weights):
    pad_n = NPAD - N
    geom2 = jnp.pad(geom, ((0, 0), (0, pad_n), (0, 0))).reshape(B, NPAD * C)
    gtp2 = jnp.pad(geom_template_posed,
                   ((0, 0), (0, pad_n), (0, 0))).reshape(B, NPAD * C)
    idx = nbs_idxs.astype(jnp.int32)
    idx3 = jnp.pad(idx, ((0, pad_n), (0, 0))).reshape(NCHUNKS, K, CH)
    wpad = jnp.pad(nbs_weights, ((0, pad_n), (0, 0)))

    table = _build_table(geom2, gtp2)
    partials = _loss_partials(table, idx3, wpad)
    return jnp.sum(partials) / (B * N * C)


# speedup vs baseline: 19.1454x; 1.0160x over previous
import functools

import jax
import jax.numpy as jnp
from jax import lax
from jax.experimental import pallas as pl
from jax.experimental.pallas import tpu as pltpu
from jax.experimental.pallas import tpu_sc as plsc

B, N, C, K = 4, 100000, 3, 16
L = 16
NC, NS = 2, 16
NW = NC * NS
NPAD = 102400

CH = 128
NCHUNKS = NPAD // CH
TRIPS = NCHUNKS // NW

CH1 = 320
TRIPS1 = NPAD // (CH1 * NW)

_params = pltpu.CompilerParams(needs_layout_passes=False,
                               use_tc_tiling_on_sc=False)
_mesh = plsc.VectorSubcoreMesh(core_axis_name="c", subcore_axis_name="s")


def _wid():
    return lax.axis_index("s") * NC + lax.axis_index("c")


@functools.partial(
    pl.kernel,
    mesh=_mesh,
    compiler_params=_params,
    out_type=jax.ShapeDtypeStruct((NPAD, L), jnp.float32),
    scratch_types=[
        pltpu.VMEM((B, CH1 * C), jnp.float32),
        pltpu.VMEM((B, CH1 * C), jnp.float32),
        pltpu.VMEM((B, CH1 * C), jnp.float32),
        pltpu.VMEM((B, CH1 * C), jnp.float32),
        pltpu.VMEM((CH1, L), jnp.float32),
        pltpu.VMEM((CH1, L), jnp.float32),
        pltpu.SemaphoreType.DMA,
        pltpu.SemaphoreType.DMA,
        pltpu.SemaphoreType.DMA,
        pltpu.SemaphoreType.DMA,
    ],
)
def _build_table(geom_hbm, gtp_hbm, table_hbm,
                 g0, g1, t0, t1, tab0, tab1, is0, is1, os0, os1):
    w = _wid()
    GV, TV, TAB = (g0, g1), (t0, t1), (tab0, tab1)
    ISEM, OSEM = (is0, is1), (os0, os1)

    def in_start(t, b2):
        off = (w * TRIPS1 + t) * (CH1 * C)
        for b in range(B):
            pltpu.async_copy(geom_hbm.at[b, pl.ds(off, CH1 * C)],
                             GV[b2].at[b], ISEM[b2])
            pltpu.async_copy(gtp_hbm.at[b, pl.ds(off, CH1 * C)],
                             TV[b2].at[b], ISEM[b2])

    def in_wait(b2):
        for b in range(B):
            pltpu.make_async_copy(geom_hbm.at[0, pl.ds(0, CH1 * C)],
                                  GV[b2].at[b], ISEM[b2]).wait()
            pltpu.make_async_copy(gtp_hbm.at[0, pl.ds(0, CH1 * C)],
                                  TV[b2].at[b], ISEM[b2]).wait()

    def out_start(t, b2):
        cbase = (w * TRIPS1 + t) * CH1
        pltpu.async_copy(TAB[b2], table_hbm.at[pl.ds(cbase, CH1)], OSEM[b2])

    def out_wait(b2):
        pltpu.make_async_copy(TAB[b2], table_hbm.at[pl.ds(0, CH1)],
                              OSEM[b2]).wait()

    lane = lax.iota(jnp.int32, L)
    valid = lane < (B * C)
    b_idx = jnp.where(valid, lane // C, 0)
    c_idx = jnp.where(valid, lane % C, 0)
    zero = jnp.zeros((L,), jnp.float32)

    def compute(b2):
        g_v, t_v, tab_v = GV[b2], TV[b2], TAB[b2]

        def node(i, c2):
            e = jnp.full((L,), i * C, jnp.int32) + c_idx
            g = plsc.load_gather(g_v, [b_idx, e])
            tt = plsc.load_gather(t_v, [b_idx, e])
            tab_v[i] = jnp.where(valid, g - tt, zero)
            return c2

        lax.fori_loop(0, CH1, node, 0)

    in_start(0, 0)

    def pair(p, carry):
        for b2 in (0, 1):
            t = 2 * p + b2
            in_wait(b2)

            @pl.when(t + 1 < TRIPS1)
            def _():
                in_start(t + 1, 1 - b2)

            @pl.when(t >= 2)
            def _():
                out_wait(b2)

            compute(b2)
            out_start(t, b2)
        return carry

    lax.fori_loop(0, TRIPS1 // 2, pair, 0)
    out_wait(0)
    out_wait(1)


@functools.partial(
    pl.kernel,
    mesh=_mesh,
    compiler_params=_params,
    out_type=jax.ShapeDtypeStruct((NW * L,), jnp.float32),
    scratch_types=[
        pltpu.VMEM((K, CH), jnp.int32),
        pltpu.VMEM((K, CH), jnp.int32),
        pltpu.VMEM((CH * K, L), jnp.float32),
        pltpu.VMEM((CH * K, L), jnp.float32),
        pltpu.VMEM((CH, L), jnp.float32),
        pltpu.VMEM((CH, L), jnp.float32),
        pltpu.VMEM((CH, L), jnp.float32),
        pltpu.VMEM((CH, L), jnp.float32),
        pltpu.VMEM((L,), jnp.float32),
        pltpu.SemaphoreType.DMA,
        pltpu.SemaphoreType.DMA,
        pltpu.SemaphoreType.DMA,
        pltpu.SemaphoreType.DMA,
    ],
)
def _loss_partials(table_hbm, idx3_hbm, w_hbm, out_hbm,
                   i0, i1, r0, r1, w0, w1, o0, o1, part_v,
                   gs0, gs1, as0, as1):
    w = _wid()
    IDX, ROWS, WV, OWN = (i0, i1), (r0, r1), (w0, w1), (o0, o1)
    GSEM, ASEM = (gs0, gs1), (as0, as1)

    def start(c, b2):
        crow = w * TRIPS + c
        cbase = crow * CH
        pltpu.sync_copy(idx3_hbm.at[crow], IDX[b2])
        for g in range(K):
            pltpu.async_copy(table_hbm.at[IDX[b2].at[g]],
                             ROWS[b2].at[pl.ds(g * CH, CH)], GSEM[b2])
        pltpu.async_copy(w_hbm.at[pl.ds(cbase, CH)], WV[b2], ASEM[b2])
        pltpu.async_copy(table_hbm.at[pl.ds(cbase, CH)], OWN[b2], ASEM[b2])

    def wait(b2):
        for g in range(K):
            pltpu.make_async_copy(table_hbm.at[pl.ds(0, CH)],
                                  ROWS[b2].at[pl.ds(g * CH, CH)],
                                  GSEM[b2]).wait()
        pltpu.make_async_copy(table_hbm.at[pl.ds(0, CH)], WV[b2],
                              ASEM[b2]).wait()
        pltpu.make_async_copy(table_hbm.at[pl.ds(0, CH)], OWN[b2],
                              ASEM[b2]).wait()

    def compute(b2, carry):
        rows, wvr, own = ROWS[b2], WV[b2], OWN[b2]

        def one(i, s):
            own_row = own[i]
            wv = wvr[i]
            base = i * K
            a0 = own_row + wv[0] * rows[base]
            a1 = wv[1] * rows[base + 1]
            a2 = wv[2] * rows[base + 2]
            a3 = wv[3] * rows[base + 3]
            for k in range(4, K, 4):
                a0 = a0 + wv[k] * rows[base + k]
                a1 = a1 + wv[k + 1] * rows[base + k + 1]
                a2 = a2 + wv[k + 2] * rows[base + k + 2]
                a3 = a3 + wv[k + 3] * rows[base + k + 3]
            acc = (a0 + a1) + (a2 + a3)
            return s + acc * acc

        def quad(q, cr):
            s0, s1, s2, s3 = cr
            i0q = q * 4
            return (one(i0q, s0), one(i0q + 1, s1),
                    one(i0q + 2, s2), one(i0q + 3, s3))

        return lax.fori_loop(0, CH // 4, quad, carry)

    start(0, 0)
    zero = jnp.zeros((L,), jnp.float32)
    carry = (zero, zero, zero, zero)

    def pair(p, cr):
        for b2 in (0, 1):
            c = 2 * p + b2
            wait(b2)
            start(c + 1, 1 - b2)
            cr = compute(b2, cr)
        return cr

    carry = lax.fori_loop(0, (TRIPS - 1) // 2, pair, carry)
    wait(0)
    s0, s1, s2, s3 = compute(0, carry)
    part_v[...] = (s0 + s1) + (s2 + s3)
    pltpu.sync_copy(part_v, out_hbm.at[pl.ds(w * L, L)])


def kernel(geom, geom_template_posed, nbs_idxs, nbs_weights):
    pad_n = NPAD - N
    geom2 = jnp.pad(geom, ((0, 0), (0, pad_n), (0, 0))).reshape(B, NPAD * C)
    gtp2 = jnp.pad(geom_template_posed,
                   ((0, 0), (0, pad_n), (0, 0))).reshape(B, NPAD * C)
    idx = nbs_idxs.astype(jnp.int32)
    idx3 = jnp.pad(idx, ((0, pad_n), (0, 0))).reshape(NCHUNKS, K, CH)
    wpad = jnp.pad(nbs_weights, ((0, pad_n), (0, 0)))

    table = _build_table(geom2, gtp2)
    partials = _loss_partials(table, idx3, wpad)
    return jnp.sum(partials) / (B * N * C)


# speedup vs baseline: 25.7174x; 1.3433x over previous
import functools

import jax
import jax.numpy as jnp
from jax import lax
from jax.experimental import pallas as pl
from jax.experimental.pallas import tpu as pltpu
from jax.experimental.pallas import tpu_sc as plsc

B, N, C, K = 4, 100000, 3, 16
L = 16
NC, NS = 2, 16
NW = NC * NS
NPAD = 102400

CH1 = 400
NCH1 = NPAD // CH1
REAL1 = N // CH1
TRIPS1 = NCH1 // NW

CH = 128
TRIPS = NPAD // (CH * NW)
FULL2 = N // CH
TAIL2 = N % CH

_params = pltpu.CompilerParams(needs_layout_passes=False,
                               use_tc_tiling_on_sc=False)
_mesh = plsc.VectorSubcoreMesh(core_axis_name="c", subcore_axis_name="s")


def _wid():
    return lax.axis_index("s") * NC + lax.axis_index("c")


@functools.partial(
    pl.kernel,
    mesh=_mesh,
    compiler_params=_params,
    out_type=jax.ShapeDtypeStruct((NPAD, L), jnp.float32),
    scratch_types=[
        pltpu.VMEM((B, CH1, C), jnp.float32),
        pltpu.VMEM((B, CH1, C), jnp.float32),
        pltpu.VMEM((B, CH1, C), jnp.float32),
        pltpu.VMEM((B, CH1, C), jnp.float32),
        pltpu.VMEM((CH1, L), jnp.float32),
        pltpu.VMEM((CH1, L), jnp.float32),
        pltpu.SemaphoreType.DMA,
        pltpu.SemaphoreType.DMA,
        pltpu.SemaphoreType.DMA,
        pltpu.SemaphoreType.DMA,
    ],
)
def _build_table(geom_hbm, gtp_hbm, table_hbm,
                 ga0, ga1, ta0, ta1, tab0, tab1, is0, is1, os0, os1):
    wid = _wid()
    GA, TA, TAB = (ga0, ga1), (ta0, ta1), (tab0, tab1)
    ISEM, OSEM = (is0, is1), (os0, os1)
    zero = jnp.zeros((L,), jnp.float32)

    def cid_of(t):
        return wid + t * NW

    def in_start(t, b2):
        cid = cid_of(t)

        @pl.when(cid < REAL1)
        def _():
            cbase = cid * CH1
            for b in range(B):
                pltpu.async_copy(geom_hbm.at[b, pl.ds(cbase, CH1), :],
                                 GA[b2].at[b], ISEM[b2])
                pltpu.async_copy(gtp_hbm.at[b, pl.ds(cbase, CH1), :],
                                 TA[b2].at[b], ISEM[b2])

    def in_wait(t, b2):
        cid = cid_of(t)

        @pl.when(cid < REAL1)
        def _():
            for b in range(B):
                pltpu.make_async_copy(geom_hbm.at[0, pl.ds(0, CH1), :],
                                      GA[b2].at[b], ISEM[b2]).wait()
                pltpu.make_async_copy(gtp_hbm.at[0, pl.ds(0, CH1), :],
                                      TA[b2].at[b], ISEM[b2]).wait()

    def out_start(t, b2):
        cbase = cid_of(t) * CH1
        pltpu.async_copy(TAB[b2], table_hbm.at[pl.ds(cbase, CH1)], OSEM[b2])

    def out_wait(b2):
        pltpu.make_async_copy(TAB[b2], table_hbm.at[pl.ds(0, CH1)],
                              OSEM[b2]).wait()

    lane = lax.iota(jnp.int32, L)
    valid = lane < (B * C)
    b_idx = jnp.where(valid, lane // C, 0)
    c_idx = jnp.where(valid, lane % C, 0)

    def compute(t, b2):
        cid = cid_of(t)
        g_v, t_v, tab_v = GA[b2], TA[b2], TAB[b2]

        @pl.when(cid < REAL1)
        def _():
            def node(i, c2):
                ii = jnp.full((L,), i, jnp.int32)
                g = plsc.load_gather(g_v, [b_idx, ii, c_idx])
                tt = plsc.load_gather(t_v, [b_idx, ii, c_idx])
                tab_v[i] = jnp.where(valid, g - tt, zero)
                return c2

            lax.fori_loop(0, CH1, node, 0)

        @pl.when(cid >= REAL1)
        def _():
            def znode(i, c2):
                tab_v[i] = zero
                return c2

            lax.fori_loop(0, CH1, znode, 0)

    in_start(0, 0)

    def pair(p, carry):
        for b2 in (0, 1):
            t = 2 * p + b2
            in_wait(t, b2)

            @pl.when(t + 1 < TRIPS1)
            def _():
                in_start(t + 1, 1 - b2)

            @pl.when(t >= 2)
            def _():
                out_wait(b2)

            compute(t, b2)
            out_start(t, b2)
        return carry

    lax.fori_loop(0, TRIPS1 // 2, pair, 0)
    out_wait(0)
    out_wait(1)


@functools.partial(
    pl.kernel,
    mesh=_mesh,
    compiler_params=_params,
    out_type=jax.ShapeDtypeStruct((NW * L,), jnp.float32),
    scratch_types=[
        pltpu.VMEM((CH, K), jnp.int32),
        pltpu.VMEM((CH, K), jnp.int32),
        pltpu.VMEM((CH * K,), jnp.int32),
        pltpu.VMEM((CH * K,), jnp.int32),
        pltpu.VMEM((CH * K, L), jnp.float32),
        pltpu.VMEM((CH * K, L), jnp.float32),
        pltpu.VMEM((CH, L), jnp.float32),
        pltpu.VMEM((CH, L), jnp.float32),
        pltpu.VMEM((CH, L), jnp.float32),
        pltpu.VMEM((CH, L), jnp.float32),
        pltpu.VMEM((L,), jnp.float32),
        pltpu.SemaphoreType.DMA,
        pltpu.SemaphoreType.DMA,
        pltpu.SemaphoreType.DMA,
        pltpu.SemaphoreType.DMA,
    ],
)
def _loss_partials(table_hbm, idx_hbm, w_hbm, out_hbm,
                   it0, it1, if0, if1, r0, r1, w0, w1, o0, o1, part_v,
                   gs0, gs1, ps0, ps1):
    sid = lax.axis_index("s")
    wid = sid * NC + lax.axis_index("c")
    IT, IF, WV, OWN = (it0, it1), (if0, if1), (w0, w1), (o0, o1)
    ROWS = (r0, r1)
    GSEM, PSEM = (gs0, gs1), (ps0, ps1)
    zero = jnp.zeros((L,), jnp.float32)
    izero = jnp.zeros((L,), jnp.int32)

    def zinit(j, c2):
        if0[pl.ds(j * L, L)] = izero
        if1[pl.ds(j * L, L)] = izero
        return c2

    lax.fori_loop(0, CH * K // L, zinit, 0)

    def crow_of(c):
        return wid * TRIPS + c

    def b_start(c, b2):
        crow = crow_of(c)
        cbase = crow * CH

        @pl.when(crow < FULL2)
        def _():
            pltpu.sync_copy(idx_hbm.at[pl.ds(cbase, CH), :], IT[b2])

            def flat(j, c2):
                IF[b2][pl.ds(j * K, K)] = IT[b2][j]
                return c2

            lax.fori_loop(0, CH, flat, 0)

        @pl.when(crow == FULL2)
        def _():
            pltpu.sync_copy(idx_hbm.at[pl.ds(N - TAIL2, TAIL2), :],
                            IT[b2].at[pl.ds(0, TAIL2), :])

            def flat(j, c2):
                IF[b2][pl.ds(j * K, K)] = IT[b2][j]
                return c2

            lax.fori_loop(0, TAIL2, flat, 0)

        for g in range(K):
            pltpu.async_copy(table_hbm.at[IF[b2].at[pl.ds(g * CH, CH)]],
                             ROWS[b2].at[pl.ds(g * CH, CH)], GSEM[b2])

        @pl.when(crow < FULL2)
        def _():
            pltpu.async_copy(w_hbm.at[pl.ds(cbase, CH), :], WV[b2], PSEM[b2])

        @pl.when(crow == FULL2)
        def _():
            pltpu.async_copy(w_hbm.at[pl.ds(N - TAIL2, TAIL2), :],
                             WV[b2].at[pl.ds(0, TAIL2), :], PSEM[b2])

        pltpu.async_copy(table_hbm.at[pl.ds(cbase, CH)], OWN[b2], PSEM[b2])

    def b_wait(c, b2):
        crow = crow_of(c)
        for g in range(K):
            pltpu.make_async_copy(table_hbm.at[pl.ds(0, CH)],
                                  ROWS[b2].at[pl.ds(g * CH, CH)],
                                  GSEM[b2]).wait()

        @pl.when(crow < FULL2)
        def _():
            pltpu.make_async_copy(w_hbm.at[pl.ds(0, CH), :], WV[b2],
                                  PSEM[b2]).wait()

        @pl.when(crow == FULL2)
        def _():
            pltpu.make_async_copy(w_hbm.at[pl.ds(0, TAIL2), :],
                                  WV[b2].at[pl.ds(0, TAIL2), :],
                                  PSEM[b2]).wait()

        pltpu.make_async_copy(table_hbm.at[pl.ds(0, CH)], OWN[b2],
                              PSEM[b2]).wait()

    def compute(c, b2, carry):
        cbase = crow_of(c) * CH
        nvalid = jnp.clip(N - cbase, 0, CH)
        rows, wvr, own = ROWS[b2], WV[b2], OWN[b2]

        def one(i, s):
            own_row = own[i]
            wv = wvr[i]
            base = i * K
            a0 = own_row + wv[0] * rows[base]
            a1 = wv[1] * rows[base + 1]
            a2 = wv[2] * rows[base + 2]
            a3 = wv[3] * rows[base + 3]
            for k in range(4, K, 4):
                a0 = a0 + wv[k] * rows[base + k]
                a1 = a1 + wv[k + 1] * rows[base + k + 1]
                a2 = a2 + wv[k + 2] * rows[base + k + 2]
                a3 = a3 + wv[k + 3] * rows[base + k + 3]
            acc = (a0 + a1) + (a2 + a3)
            return s + acc * acc

        def quad(q, cr):
            s0, s1, s2, s3 = cr
            i0q = q * 4
            return (one(i0q, s0), one(i0q + 1, s1),
                    one(i0q + 2, s2), one(i0q + 3, s3))

        return lax.fori_loop(0, nvalid // 4, quad, carry)

    b_start(0, 0)
    carry = (zero, zero, zero, zero)

    def pair(p, cr):
        for b2 in (0, 1):
            c = 2 * p + b2
            b_wait(c, b2)
            b_start(c + 1, 1 - b2)
            cr = compute(c, b2, cr)
        return cr

    carry = lax.fori_loop(0, (TRIPS - 1) // 2, pair, carry)
    b_wait(TRIPS - 1, 0)
    s0, s1, s2, s3 = compute(TRIPS - 1, 0, carry)
    part_v[...] = (s0 + s1) + (s2 + s3)
    pltpu.sync_copy(part_v, out_hbm.at[pl.ds(wid * L, L)])


def kernel(geom, geom_template_posed, nbs_idxs, nbs_weights):
    idx = nbs_idxs.astype(jnp.int32)
    table = _build_table(geom, geom_template_posed)
    partials = _loss_partials(table, idx, nbs_weights)
    return jnp.sum(partials) / (B * N * C)


# speedup vs baseline: 41.8170x; 1.6260x over previous
import functools

import jax
import jax.numpy as jnp
from jax import lax
from jax.experimental import pallas as pl
from jax.experimental.pallas import tpu as pltpu
from jax.experimental.pallas import tpu_sc as plsc

B, N, C, K = 4, 100000, 3, 16
L = 16
NC, NS = 2, 16
NW = NC * NS
NPAD = 102400

CH1 = 400
NCH1 = NPAD // CH1
REAL1 = N // CH1
TRIPS1 = NCH1 // NW

CH = 128
TRIPS = NPAD // (CH * NW)
FULL2 = N // CH
TAIL2 = N % CH

_params = pltpu.CompilerParams(needs_layout_passes=False,
                               use_tc_tiling_on_sc=False)
_params_tiled = pltpu.CompilerParams(needs_layout_passes=False,
                                     use_tc_tiling_on_sc=True)
_mesh = plsc.VectorSubcoreMesh(core_axis_name="c", subcore_axis_name="s")


def _wid():
    return lax.axis_index("s") * NC + lax.axis_index("c")


@functools.partial(
    pl.kernel,
    mesh=_mesh,
    compiler_params=_params,
    out_type=jax.ShapeDtypeStruct((NPAD, L), jnp.float32),
    scratch_types=[
        pltpu.VMEM((B, CH1, C), jnp.float32),
        pltpu.VMEM((B, CH1, C), jnp.float32),
        pltpu.VMEM((CH1, L), jnp.float32),
        pltpu.VMEM((CH1, L), jnp.float32),
        pltpu.SemaphoreType.DMA,
        pltpu.SemaphoreType.DMA,
        pltpu.SemaphoreType.DMA,
        pltpu.SemaphoreType.DMA,
    ],
)
def _build_table(d_hbm, table_hbm,
                 ga0, ga1, tab0, tab1, is0, is1, os0, os1):
    wid = _wid()
    GA, TAB = (ga0, ga1), (tab0, tab1)
    ISEM, OSEM = (is0, is1), (os0, os1)
    zero = jnp.zeros((L,), jnp.float32)

    def cid_of(t):
        return wid + t * NW

    def in_start(t, b2):
        cid = cid_of(t)

        @pl.when(cid < REAL1)
        def _():
            cbase = cid * CH1
            for b in range(B):
                pltpu.async_copy(d_hbm.at[b, pl.ds(cbase, CH1), :],
                                 GA[b2].at[b], ISEM[b2])

    def in_wait(t, b2):
        cid = cid_of(t)

        @pl.when(cid < REAL1)
        def _():
            for b in range(B):
                pltpu.make_async_copy(d_hbm.at[0, pl.ds(0, CH1), :],
                                      GA[b2].at[b], ISEM[b2]).wait()

    def out_start(t, b2):
        cbase = cid_of(t) * CH1
        pltpu.async_copy(TAB[b2], table_hbm.at[pl.ds(cbase, CH1)], OSEM[b2])

    def out_wait(b2):
        pltpu.make_async_copy(TAB[b2], table_hbm.at[pl.ds(0, CH1)],
                              OSEM[b2]).wait()

    lane = lax.iota(jnp.int32, L)
    valid = lane < (B * C)
    b_idx = jnp.where(valid, lane // C, 0)
    c_idx = jnp.where(valid, lane % C, 0)

    def compute(t, b2):
        cid = cid_of(t)
        g_v, tab_v = GA[b2], TAB[b2]

        @pl.when(cid < REAL1)
        def _():
            def node(i, c2):
                ii = jnp.full((L,), i, jnp.int32)
                g = plsc.load_gather(g_v, [b_idx, ii, c_idx])
                tab_v[i] = jnp.where(valid, g, zero)
                return c2

            lax.fori_loop(0, CH1, node, 0)

        @pl.when(cid >= REAL1)
        def _():
            def znode(i, c2):
                tab_v[i] = zero
                return c2

            lax.fori_loop(0, CH1, znode, 0)

    in_start(0, 0)

    def pair(p, carry):
        for b2 in (0, 1):
            t = 2 * p + b2
            in_wait(t, b2)

            @pl.when(t + 1 < TRIPS1)
            def _():
                in_start(t + 1, 1 - b2)

            @pl.when(t >= 2)
            def _():
                out_wait(b2)

            compute(t, b2)
            out_start(t, b2)
        return carry

    lax.fori_loop(0, TRIPS1 // 2, pair, 0)
    out_wait(0)
    out_wait(1)


@functools.partial(
    pl.kernel,
    mesh=_mesh,
    compiler_params=_params,
    out_type=jax.ShapeDtypeStruct((NW * L,), jnp.float32),
    scratch_types=[
        pltpu.VMEM((CH, K), jnp.int32),
        pltpu.VMEM((CH, K), jnp.int32),
        pltpu.VMEM((CH * K,), jnp.int32),
        pltpu.VMEM((CH * K,), jnp.int32),
        pltpu.VMEM((CH * K, L), jnp.float32),
        pltpu.VMEM((CH * K, L), jnp.float32),
        pltpu.VMEM((CH, L), jnp.float32),
        pltpu.VMEM((CH, L), jnp.float32),
        pltpu.VMEM((CH, L), jnp.float32),
        pltpu.VMEM((CH, L), jnp.float32),
        pltpu.VMEM((L,), jnp.float32),
        pltpu.SemaphoreType.DMA,
        pltpu.SemaphoreType.DMA,
        pltpu.SemaphoreType.DMA,
        pltpu.SemaphoreType.DMA,
    ],
)
def _loss_partials(table_hbm, idx_hbm, w_hbm, out_hbm,
                   it0, it1, if0, if1, r0, r1, w0, w1, o0, o1, part_v,
                   gs0, gs1, ps0, ps1):
    sid = lax.axis_index("s")
    wid = sid * NC + lax.axis_index("c")
    IT, IF, WV, OWN = (it0, it1), (if0, if1), (w0, w1), (o0, o1)
    ROWS = (r0, r1)
    GSEM, PSEM = (gs0, gs1), (ps0, ps1)
    zero = jnp.zeros((L,), jnp.float32)
    izero = jnp.zeros((L,), jnp.int32)

    def zinit(j, c2):
        if0[pl.ds(j * L, L)] = izero
        if1[pl.ds(j * L, L)] = izero
        return c2

    lax.fori_loop(0, CH * K // L, zinit, 0)

    def crow_of(c):
        return wid * TRIPS + c

    def b_start(c, b2):
        crow = crow_of(c)
        cbase = crow * CH

        @pl.when(crow < FULL2)
        def _():
            pltpu.sync_copy(idx_hbm.at[pl.ds(cbase, CH), :], IT[b2])

            def flat(j, c2):
                IF[b2][pl.ds(j * K, K)] = IT[b2][j]
                return c2

            lax.fori_loop(0, CH, flat, 0)

        @pl.when(crow == FULL2)
        def _():
            pltpu.sync_copy(idx_hbm.at[pl.ds(N - TAIL2, TAIL2), :],
                            IT[b2].at[pl.ds(0, TAIL2), :])

            def flat(j, c2):
                IF[b2][pl.ds(j * K, K)] = IT[b2][j]
                return c2

            lax.fori_loop(0, TAIL2, flat, 0)

        for g in range(K):
            pltpu.async_copy(table_hbm.at[IF[b2].at[pl.ds(g * CH, CH)]],
                             ROWS[b2].at[pl.ds(g * CH, CH)], GSEM[b2])

        @pl.when(crow < FULL2)
        def _():
            pltpu.async_copy(w_hbm.at[pl.ds(cbase, CH), :], WV[b2], PSEM[b2])

        @pl.when(crow == FULL2)
        def _():
            pltpu.async_copy(w_hbm.at[pl.ds(N - TAIL2, TAIL2), :],
                             WV[b2].at[pl.ds(0, TAIL2), :], PSEM[b2])

        pltpu.async_copy(table_hbm.at[pl.ds(cbase, CH)], OWN[b2], PSEM[b2])

    def b_wait(c, b2):
        crow = crow_of(c)
        for g in range(K):
            pltpu.make_async_copy(table_hbm.at[pl.ds(0, CH)],
                                  ROWS[b2].at[pl.ds(g * CH, CH)],
                                  GSEM[b2]).wait()

        @pl.when(crow < FULL2)
        def _():
            pltpu.make_async_copy(w_hbm.at[pl.ds(0, CH), :], WV[b2],
                                  PSEM[b2]).wait()

        @pl.when(crow == FULL2)
        def _():
            pltpu.make_async_copy(w_hbm.at[pl.ds(0, TAIL2), :],
                                  WV[b2].at[pl.ds(0, TAIL2), :],
                                  PSEM[b2]).wait()

        pltpu.make_async_copy(table_hbm.at[pl.ds(0, CH)], OWN[b2],
                              PSEM[b2]).wait()

    def compute(c, b2, carry):
        cbase = crow_of(c) * CH
        nvalid = jnp.clip(N - cbase, 0, CH)
        rows, wvr, own = ROWS[b2], WV[b2], OWN[b2]

        def one(i, s):
            own_row = own[i]
            wv = wvr[i]
            base = i * K
            a0 = own_row + wv[0] * rows[base]
            a1 = wv[1] * rows[base + 1]
            a2 = wv[2] * rows[base + 2]
            a3 = wv[3] * rows[base + 3]
            for k in range(4, K, 4):
                a0 = a0 + wv[k] * rows[base + k]
                a1 = a1 + wv[k + 1] * rows[base + k + 1]
                a2 = a2 + wv[k + 2] * rows[base + k + 2]
                a3 = a3 + wv[k + 3] * rows[base + k + 3]
            acc = (a0 + a1) + (a2 + a3)
            return s + acc * acc

        def quad(q, cr):
            s0, s1, s2, s3 = cr
            i0q = q * 4
            return (one(i0q, s0), one(i0q + 1, s1),
                    one(i0q + 2, s2), one(i0q + 3, s3))

        return lax.fori_loop(0, nvalid // 4, quad, carry)

    b_start(0, 0)
    carry = (zero, zero, zero, zero)

    def pair(p, cr):
        for b2 in (0, 1):
            c = 2 * p + b2
            b_wait(c, b2)
            b_start(c + 1, 1 - b2)
            cr = compute(c, b2, cr)
        return cr

    carry = lax.fori_loop(0, (TRIPS - 1) // 2, pair, carry)
    b_wait(TRIPS - 1, 0)
    s0, s1, s2, s3 = compute(TRIPS - 1, 0, carry)
    part_v[...] = (s0 + s1) + (s2 + s3)
    pltpu.sync_copy(part_v, out_hbm.at[pl.ds(wid * L, L)])


def kernel(geom, geom_template_posed, nbs_idxs, nbs_weights):
    idx = nbs_idxs.astype(jnp.int32)
    d = geom - geom_template_posed
    table = _build_table(d)
    partials = _loss_partials(table, idx, nbs_weights)
    return jnp.sum(partials) / (B * N * C)


# speedup vs baseline: 52.7628x; 1.2618x over previous
import functools

import jax
import jax.numpy as jnp
from jax import lax
from jax.experimental import pallas as pl
from jax.experimental.pallas import tpu as pltpu
from jax.experimental.pallas import tpu_sc as plsc

B, N, C, K = 4, 100000, 3, 16
L = 16
NC, NS = 2, 16
NW = NC * NS
NPAD = 102400

CH1 = 400
NCH1 = NPAD // CH1
REAL1 = N // CH1
TRIPS1 = NCH1 // NW

CH = 128
TRIPS = NPAD // (CH * NW)
FULL2 = N // CH
TAIL2 = N % CH

_params = pltpu.CompilerParams(needs_layout_passes=False,
                               use_tc_tiling_on_sc=False)
_params_tiled = pltpu.CompilerParams(needs_layout_passes=False,
                                     use_tc_tiling_on_sc=True)
_mesh = plsc.VectorSubcoreMesh(core_axis_name="c", subcore_axis_name="s")


def _wid():
    return lax.axis_index("s") * NC + lax.axis_index("c")


@functools.partial(
    pl.kernel,
    mesh=_mesh,
    compiler_params=_params,
    out_type=jax.ShapeDtypeStruct((NPAD, L), jnp.float32),
    scratch_types=[
        pltpu.VMEM((B, CH1 * C), jnp.float32),
        pltpu.VMEM((B, CH1 * C), jnp.float32),
        pltpu.VMEM((CH1, L), jnp.float32),
        pltpu.VMEM((CH1, L), jnp.float32),
        pltpu.SemaphoreType.DMA,
        pltpu.SemaphoreType.DMA,
        pltpu.SemaphoreType.DMA,
        pltpu.SemaphoreType.DMA,
    ],
)
def _build_table(d_hbm, table_hbm,
                 ga0, ga1, tab0, tab1, is0, is1, os0, os1):
    wid = _wid()
    GA, TAB = (ga0, ga1), (tab0, tab1)
    ISEM, OSEM = (is0, is1), (os0, os1)
    zero = jnp.zeros((L,), jnp.float32)

    def cid_of(t):
        return wid + t * NW

    def in_start(t, b2):
        cid = cid_of(t)

        @pl.when(cid < REAL1)
        def _():
            cbase = cid * CH1
            for b in range(B):
                pltpu.async_copy(d_hbm.at[b, pl.ds(cbase * C, CH1 * C)],
                                 GA[b2].at[b], ISEM[b2])

    def in_wait(t, b2):
        cid = cid_of(t)

        @pl.when(cid < REAL1)
        def _():
            for b in range(B):
                pltpu.make_async_copy(d_hbm.at[0, pl.ds(0, CH1 * C)],
                                      GA[b2].at[b], ISEM[b2]).wait()

    def out_start(t, b2):
        cbase = cid_of(t) * CH1
        pltpu.async_copy(TAB[b2], table_hbm.at[pl.ds(cbase, CH1)], OSEM[b2])

    def out_wait(b2):
        pltpu.make_async_copy(TAB[b2], table_hbm.at[pl.ds(0, CH1)],
                              OSEM[b2]).wait()

    lane = lax.iota(jnp.int32, L)
    valid = lane < (B * C)
    b_idx = jnp.where(valid, lane // C, 0)
    c_idx = jnp.where(valid, lane % C, 0)

    def compute(t, b2):
        cid = cid_of(t)
        g_v, tab_v = GA[b2], TAB[b2]

        @pl.when(cid < REAL1)
        def _():
            def node(i, c2):
                e = jnp.full((L,), i * C, jnp.int32) + c_idx
                g = plsc.load_gather(g_v, [b_idx, e])
                tab_v[i] = jnp.where(valid, g, zero)
                return c2

            lax.fori_loop(0, CH1, node, 0)

        @pl.when(cid >= REAL1)
        def _():
            def znode(i, c2):
                tab_v[i] = zero
                return c2

            lax.fori_loop(0, CH1, znode, 0)

    in_start(0, 0)

    def pair(p, carry):
        for b2 in (0, 1):
            t = 2 * p + b2
            in_wait(t, b2)

            @pl.when(t + 1 < TRIPS1)
            def _():
                in_start(t + 1, 1 - b2)

            @pl.when(t >= 2)
            def _():
                out_wait(b2)

            compute(t, b2)
            out_start(t, b2)
        return carry

    lax.fori_loop(0, TRIPS1 // 2, pair, 0)
    out_wait(0)
    out_wait(1)


@functools.partial(
    pl.kernel,
    mesh=_mesh,
    compiler_params=_params,
    out_type=jax.ShapeDtypeStruct((NW * L,), jnp.float32),
    scratch_types=[
        pltpu.VMEM((CH, K), jnp.int32),
        pltpu.VMEM((CH, K), jnp.int32),
        pltpu.VMEM((CH * K,), jnp.int32),
        pltpu.VMEM((CH * K,), jnp.int32),
        pltpu.VMEM((CH * K, L), jnp.float32),
        pltpu.VMEM((CH * K, L), jnp.float32),
        pltpu.VMEM((CH, L), jnp.float32),
        pltpu.VMEM((CH, L), jnp.float32),
        pltpu.VMEM((CH, L), jnp.float32),
        pltpu.VMEM((CH, L), jnp.float32),
        pltpu.VMEM((L,), jnp.float32),
        pltpu.SemaphoreType.DMA,
        pltpu.SemaphoreType.DMA,
        pltpu.SemaphoreType.DMA,
        pltpu.SemaphoreType.DMA,
    ],
)
def _loss_partials(table_hbm, idx_hbm, w_hbm, out_hbm,
                   it0, it1, if0, if1, r0, r1, w0, w1, o0, o1, part_v,
                   gs0, gs1, ps0, ps1):
    sid = lax.axis_index("s")
    wid = sid * NC + lax.axis_index("c")
    IT, IF, WV, OWN = (it0, it1), (if0, if1), (w0, w1), (o0, o1)
    ROWS = (r0, r1)
    GSEM, PSEM = (gs0, gs1), (ps0, ps1)
    zero = jnp.zeros((L,), jnp.float32)
    izero = jnp.zeros((L,), jnp.int32)

    def zinit(j, c2):
        if0[pl.ds(j * L, L)] = izero
        if1[pl.ds(j * L, L)] = izero
        return c2

    lax.fori_loop(0, CH * K // L, zinit, 0)

    def crow_of(c):
        return wid * TRIPS + c

    def b_start(c, b2):
        crow = crow_of(c)
        cbase = crow * CH

        @pl.when(crow < FULL2)
        def _():
            pltpu.sync_copy(idx_hbm.at[pl.ds(cbase, CH), :], IT[b2])

            def flat(j, c2):
                IF[b2][pl.ds(j * K, K)] = IT[b2][j]
                return c2

            lax.fori_loop(0, CH, flat, 0)

        @pl.when(crow == FULL2)
        def _():
            pltpu.sync_copy(idx_hbm.at[pl.ds(N - TAIL2, TAIL2), :],
                            IT[b2].at[pl.ds(0, TAIL2), :])

            def flat(j, c2):
                IF[b2][pl.ds(j * K, K)] = IT[b2][j]
                return c2

            lax.fori_loop(0, TAIL2, flat, 0)

        for g in range(K):
            pltpu.async_copy(table_hbm.at[IF[b2].at[pl.ds(g * CH, CH)]],
                             ROWS[b2].at[pl.ds(g * CH, CH)], GSEM[b2])

        @pl.when(crow < FULL2)
        def _():
            pltpu.async_copy(w_hbm.at[pl.ds(cbase, CH), :], WV[b2], PSEM[b2])

        @pl.when(crow == FULL2)
        def _():
            pltpu.async_copy(w_hbm.at[pl.ds(N - TAIL2, TAIL2), :],
                             WV[b2].at[pl.ds(0, TAIL2), :], PSEM[b2])

        pltpu.async_copy(table_hbm.at[pl.ds(cbase, CH)], OWN[b2], PSEM[b2])

    def b_wait(c, b2):
        crow = crow_of(c)
        for g in range(K):
            pltpu.make_async_copy(table_hbm.at[pl.ds(0, CH)],
                                  ROWS[b2].at[pl.ds(g * CH, CH)],
                                  GSEM[b2]).wait()

        @pl.when(crow < FULL2)
        def _():
            pltpu.make_async_copy(w_hbm.at[pl.ds(0, CH), :], WV[b2],
                                  PSEM[b2]).wait()

        @pl.when(crow == FULL2)
        def _():
            pltpu.make_async_copy(w_hbm.at[pl.ds(0, TAIL2), :],
                                  WV[b2].at[pl.ds(0, TAIL2), :],
                                  PSEM[b2]).wait()

        pltpu.make_async_copy(table_hbm.at[pl.ds(0, CH)], OWN[b2],
                              PSEM[b2]).wait()

    def compute(c, b2, carry):
        cbase = crow_of(c) * CH
        nvalid = jnp.clip(N - cbase, 0, CH)
        rows, wvr, own = ROWS[b2], WV[b2], OWN[b2]

        def one(i, s):
            own_row = own[i]
            wv = wvr[i]
            base = i * K
            a0 = own_row + wv[0] * rows[base]
            a1 = wv[1] * rows[base + 1]
            a2 = wv[2] * rows[base + 2]
            a3 = wv[3] * rows[base + 3]
            for k in range(4, K, 4):
                a0 = a0 + wv[k] * rows[base + k]
                a1 = a1 + wv[k + 1] * rows[base + k + 1]
                a2 = a2 + wv[k + 2] * rows[base + k + 2]
                a3 = a3 + wv[k + 3] * rows[base + k + 3]
            acc = (a0 + a1) + (a2 + a3)
            return s + acc * acc

        def quad(q, cr):
            s0, s1, s2, s3 = cr
            i0q = q * 4
            return (one(i0q, s0), one(i0q + 1, s1),
                    one(i0q + 2, s2), one(i0q + 3, s3))

        return lax.fori_loop(0, nvalid // 4, quad, carry)

    b_start(0, 0)
    carry = (zero, zero, zero, zero)

    def pair(p, cr):
        for b2 in (0, 1):
            c = 2 * p + b2
            b_wait(c, b2)
            b_start(c + 1, 1 - b2)
            cr = compute(c, b2, cr)
        return cr

    carry = lax.fori_loop(0, (TRIPS - 1) // 2, pair, carry)
    b_wait(TRIPS - 1, 0)
    s0, s1, s2, s3 = compute(TRIPS - 1, 0, carry)
    part_v[...] = (s0 + s1) + (s2 + s3)
    pltpu.sync_copy(part_v, out_hbm.at[pl.ds(wid * L, L)])


def kernel(geom, geom_template_posed, nbs_idxs, nbs_weights):
    idx = nbs_idxs.astype(jnp.int32)
    d = (geom - geom_template_posed).reshape(B, N * C)
    table = _build_table(d)
    partials = _loss_partials(table, idx, nbs_weights)
    return jnp.sum(partials) / (B * N * C)
